# baseline (speedup 1.0000x reference)
import functools

import jax
import jax.numpy as jnp
from jax import lax
from jax.experimental import pallas as pl
from jax.experimental.pallas import tpu as pltpu

_THRESH = -3.0
_BM = 256


def _pass1_body(dA_ref, dB_ref, mA_ref, mB_ref,
                scores_ref, rmax_ref, rarg_ref, cmax_ref,
                dBn_scr, cmax_scr, *, nm, precision):
    i = pl.program_id(1)

    @pl.when(i == 0)
    def _():
        dB = dB_ref[0]
        nB = jnp.sqrt(jnp.sum(dB * dB, axis=-1, keepdims=True))
        dBn_scr[...] = dB / (nB + 1e-12)

    dA = dA_ref[0]
    nA = jnp.sqrt(jnp.sum(dA * dA, axis=-1, keepdims=True))
    dAn = dA / (nA + 1e-12)
    d = dA.shape[-1]
    s = lax.dot_general(dAn, dBn_scr[...], (((1,), (1,)), ((), ())),
                        precision=precision,
                        preferred_element_type=jnp.float32)
    s = s * (float(d) ** 0.5)
    s = (s + mA_ref[0, 0][:, None]) + mB_ref[0, 0][None, :]
    scores_ref[0] = s

    n = s.shape[1]
    rmax = jnp.max(s, axis=1)
    rmax_ref[0, 0] = rmax
    eqf = jnp.where(s == rmax[:, None], 1.0, 0.0)
    iota_col = lax.broadcasted_iota(jnp.int32, (n, 1), 0).astype(jnp.float32)
    rarg_f = lax.dot_general(eqf, iota_col, (((1,), (0,)), ((), ())),
                             precision=lax.Precision.HIGHEST,
                             preferred_element_type=jnp.float32)
    rarg_ref[0, 0] = rarg_f[:, 0].astype(jnp.int32)

    tmax = jnp.max(s, axis=0, keepdims=True)
    cmax_scr[...] = jnp.where(i == 0, tmax,
                              jnp.maximum(cmax_scr[...], tmax))

    @pl.when(i == nm - 1)
    def _():
        cmax_ref[0] = cmax_scr[...]


def _pass2_body(rmax_ref, rarg_ref, cmax_ref, conf_ref):
    rm = rmax_ref[0, 0]
    ra = rarg_ref[0, 0]
    cm = cmax_ref[0, 0]
    bm = rm.shape[0]
    n = cm.shape[0]
    col_iota = lax.broadcasted_iota(jnp.int32, (bm, n), 1)
    mut = jnp.logical_and(col_iota == ra[:, None], rm[:, None] == cm[None, :])
    sig = jnp.where(rm >= _THRESH, jax.nn.sigmoid(rm), 0.0)
    conf_ref[0] = jnp.where(mut, jnp.broadcast_to(sig[:, None], (bm, n)), 0.0)


def kernel(desc_A, desc_B, matchability_A, matchability_B):
    B, M, D = desc_A.shape
    N = desc_B.shape[1]
    bm = _BM
    nm = M // bm
    mA3 = matchability_A.reshape(B, 1, M)
    mB3 = matchability_B.reshape(B, 1, N)

    p1 = pl.pallas_call(
        functools.partial(_pass1_body, nm=nm, precision=lax.Precision.DEFAULT),
        grid=(B, nm),
        in_specs=[
            pl.BlockSpec((1, bm, D), lambda b, i: (b, i, 0)),
            pl.BlockSpec((1, N, D), lambda b, i: (b, 0, 0)),
            pl.BlockSpec((1, 1, bm), lambda b, i: (b, 0, i)),
            pl.BlockSpec((1, 1, N), lambda b, i: (b, 0, 0)),
        ],
        out_specs=[
            pl.BlockSpec((1, bm, N), lambda b, i: (b, i, 0)),
            pl.BlockSpec((1, 1, bm), lambda b, i: (b, 0, i)),
            pl.BlockSpec((1, 1, bm), lambda b, i: (b, 0, i)),
            pl.BlockSpec((1, 1, N), lambda b, i: (b, 0, 0)),
        ],
        out_shape=[
            jax.ShapeDtypeStruct((B, M, N), jnp.float32),
            jax.ShapeDtypeStruct((B, 1, M), jnp.float32),
            jax.ShapeDtypeStruct((B, 1, M), jnp.int32),
            jax.ShapeDtypeStruct((B, 1, N), jnp.float32),
        ],
        scratch_shapes=[
            pltpu.VMEM((N, D), jnp.float32),
            pltpu.VMEM((1, N), jnp.float32),
        ],
        compiler_params=pltpu.CompilerParams(
            dimension_semantics=("arbitrary", "arbitrary")),
    )
    scores, rowmax, rowarg, colmax = p1(desc_A, desc_B, mA3, mB3)

    p2 = pl.pallas_call(
        _pass2_body,
        grid=(B, nm),
        in_specs=[
            pl.BlockSpec((1, 1, bm), lambda b, i: (b, 0, i)),
            pl.BlockSpec((1, 1, bm), lambda b, i: (b, 0, i)),
            pl.BlockSpec((1, 1, N), lambda b, i: (b, 0, 0)),
        ],
        out_specs=pl.BlockSpec((1, bm, N), lambda b, i: (b, i, 0)),
        out_shape=jax.ShapeDtypeStruct((B, M, N), jnp.float32),
        compiler_params=pltpu.CompilerParams(
            dimension_semantics=("arbitrary", "arbitrary")),
    )
    confidence = p2(rowmax, rowarg, colmax)
    return scores, confidence


# speedup vs baseline: 1.8698x; 1.8698x over previous
import functools

import jax
import jax.numpy as jnp
from jax import lax
from jax.experimental import pallas as pl
from jax.experimental.pallas import tpu as pltpu

_THRESH = -3.0


def _body(dA_ref, dB_ref, mA_ref, mB_ref, scores_ref, conf_ref,
          rmax_scr, rarg_scr, cmax0_scr, *, bn, precision):
    j = pl.program_id(1)

    @pl.when(j < 2)
    def _():
        dB = dB_ref[0]
        nB = jnp.sqrt(jnp.sum(dB * dB, axis=-1, keepdims=True))
        dBn = dB / (nB + 1e-12)

        dA = dA_ref[0]
        nA = jnp.sqrt(jnp.sum(dA * dA, axis=-1, keepdims=True))
        d = dA.shape[-1]
        dAn = (dA / (nA + 1e-12)) * (float(d) ** 0.5)

        s = lax.dot_general(dAn, dBn, (((1,), (1,)), ((), ())),
                            precision=precision,
                            preferred_element_type=jnp.float32)
        s = (s + mA_ref[0, 0][:, None]) + mB_ref[0, 0][None, :]
        scores_ref[0] = s

        m = s.shape[0]
        rmax_h = jnp.max(s, axis=1)
        col_iota = lax.broadcasted_iota(jnp.int32, (m, bn), 1)
        cand = jnp.where(s == rmax_h[:, None], col_iota, jnp.int32(2147483647))
        rarg_h = jnp.min(cand, axis=1) + j * bn
        cmax_h = jnp.max(s, axis=0)

        @pl.when(j == 0)
        def _():
            rmax_scr[...] = rmax_h[None, :]
            rarg_scr[...] = rarg_h[None, :]
            cmax0_scr[...] = cmax_h[None, :]

        @pl.when(j == 1)
        def _():
            prev_max = rmax_scr[0]
            prev_arg = rarg_scr[0]
            upd = rmax_h > prev_max
            rmax = jnp.where(upd, rmax_h, prev_max)
            rarg = jnp.where(upd, rarg_h, prev_arg)
            rmax_scr[...] = rmax[None, :]
            rarg_scr[...] = rarg[None, :]
            ci = col_iota + bn
            mut = jnp.logical_and(ci == rarg[:, None],
                                  rmax[:, None] == cmax_h[None, :])
            sig = jnp.where(rmax >= _THRESH, jax.nn.sigmoid(rmax), 0.0)
            conf_ref[0] = jnp.where(
                mut, jnp.broadcast_to(sig[:, None], (m, bn)), 0.0)

    @pl.when(j == 2)
    def _():
        rmax = rmax_scr[0]
        rarg = rarg_scr[0]
        cmax0 = cmax0_scr[0]
        m = rmax.shape[0]
        col_iota = lax.broadcasted_iota(jnp.int32, (m, bn), 1)
        mut = jnp.logical_and(col_iota == rarg[:, None],
                              rmax[:, None] == cmax0[None, :])
        sig = jnp.where(rmax >= _THRESH, jax.nn.sigmoid(rmax), 0.0)
        conf_ref[0] = jnp.where(
            mut, jnp.broadcast_to(sig[:, None], (m, bn)), 0.0)


def kernel(desc_A, desc_B, matchability_A, matchability_B):
    B, M, D = desc_A.shape
    N = desc_B.shape[1]
    bn = N // 2
    mA3 = matchability_A.reshape(B, 1, M)
    mB3 = matchability_B.reshape(B, 1, N)

    p = pl.pallas_call(
        functools.partial(_body, bn=bn, precision=lax.Precision.DEFAULT),
        grid=(B, 3),
        in_specs=[
            pl.BlockSpec((1, M, D), lambda b, j: (b, 0, 0)),
            pl.BlockSpec((1, bn, D), lambda b, j: (b, jnp.minimum(j, 1), 0)),
            pl.BlockSpec((1, 1, M), lambda b, j: (b, 0, 0)),
            pl.BlockSpec((1, 1, bn), lambda b, j: (b, 0, jnp.minimum(j, 1))),
        ],
        out_specs=[
            pl.BlockSpec((1, M, bn), lambda b, j: (b, 0, jnp.minimum(j, 1))),
            pl.BlockSpec((1, M, bn),
                         lambda b, j: (b, 0, jnp.where(j < 2, 1, 0))),
        ],
        out_shape=[
            jax.ShapeDtypeStruct((B, M, N), jnp.float32),
            jax.ShapeDtypeStruct((B, M, N), jnp.float32),
        ],
        scratch_shapes=[
            pltpu.VMEM((1, M), jnp.float32),
            pltpu.VMEM((1, M), jnp.int32),
            pltpu.VMEM((1, bn), jnp.float32),
        ],
        compiler_params=pltpu.CompilerParams(
            dimension_semantics=("arbitrary", "arbitrary")),
    )
    scores, confidence = p(desc_A, desc_B, mA3, mB3)
    return scores, confidence


# speedup vs baseline: 2.2208x; 1.1877x over previous
import functools

import jax
import jax.numpy as jnp
from jax import lax
from jax.experimental import pallas as pl
from jax.experimental.pallas import tpu as pltpu

_THRESH = -3.0


def _pass1_body(dA_ref, dB_ref, mA_ref, mB_ref,
                scores_ref, rmax_ref, rarg_ref, cmax_ref, *, precision):
    dB = dB_ref[0]
    nB = jnp.sqrt(jnp.sum(dB * dB, axis=-1, keepdims=True))
    dBn = dB / (nB + 1e-12)

    dA = dA_ref[0]
    nA = jnp.sqrt(jnp.sum(dA * dA, axis=-1, keepdims=True))
    d = dA.shape[-1]
    dAn = (dA / (nA + 1e-12)) * (float(d) ** 0.5)

    s = lax.dot_general(dAn, dBn, (((1,), (1,)), ((), ())),
                        precision=precision,
                        preferred_element_type=jnp.float32)
    s = (s + mA_ref[0, 0][:, None]) + mB_ref[0, 0][None, :]
    scores_ref[0] = s

    m, n = s.shape
    rmax = jnp.max(s, axis=1)
    rmax_ref[0, 0] = rmax
    col_iota = lax.broadcasted_iota(jnp.int32, (m, n), 1)
    cand = jnp.where(s == rmax[:, None], col_iota, jnp.int32(2147483647))
    rarg_ref[0, 0] = jnp.min(cand, axis=1)
    cmax_ref[0, 0] = jnp.max(s, axis=0)


def _pass2_body(rmax_ref, rarg_ref, cmax_ref, conf_ref):
    rm = rmax_ref[0, 0]
    ra = rarg_ref[0, 0]
    cm = cmax_ref[0, 0]
    m = rm.shape[0]
    n = cm.shape[0]
    col_iota = lax.broadcasted_iota(jnp.int32, (m, n), 1)
    mut = jnp.logical_and(col_iota == ra[:, None], rm[:, None] == cm[None, :])
    sig = jnp.where(rm >= _THRESH, jax.nn.sigmoid(rm), 0.0)
    conf_ref[0] = jnp.where(mut, jnp.broadcast_to(sig[:, None], (m, n)), 0.0)


def kernel(desc_A, desc_B, matchability_A, matchability_B):
    B, M, D = desc_A.shape
    N = desc_B.shape[1]
    mA3 = matchability_A.reshape(B, 1, M)
    mB3 = matchability_B.reshape(B, 1, N)

    p1 = pl.pallas_call(
        functools.partial(_pass1_body, precision=lax.Precision.DEFAULT),
        grid=(B,),
        in_specs=[
            pl.BlockSpec((1, M, D), lambda b: (b, 0, 0)),
            pl.BlockSpec((1, N, D), lambda b: (b, 0, 0)),
            pl.BlockSpec((1, 1, M), lambda b: (b, 0, 0)),
            pl.BlockSpec((1, 1, N), lambda b: (b, 0, 0)),
        ],
        out_specs=[
            pl.BlockSpec((1, M, N), lambda b: (b, 0, 0)),
            pl.BlockSpec((1, 1, M), lambda b: (b, 0, 0)),
            pl.BlockSpec((1, 1, M), lambda b: (b, 0, 0)),
            pl.BlockSpec((1, 1, N), lambda b: (b, 0, 0)),
        ],
        out_shape=[
            jax.ShapeDtypeStruct((B, M, N), jnp.float32),
            jax.ShapeDtypeStruct((B, 1, M), jnp.float32),
            jax.ShapeDtypeStruct((B, 1, M), jnp.int32),
            jax.ShapeDtypeStruct((B, 1, N), jnp.float32),
        ],
        compiler_params=pltpu.CompilerParams(
            dimension_semantics=("arbitrary",)),
    )
    scores, rowmax, rowarg, colmax = p1(desc_A, desc_B, mA3, mB3)

    p2 = pl.pallas_call(
        _pass2_body,
        grid=(B,),
        in_specs=[
            pl.BlockSpec((1, 1, M), lambda b: (b, 0, 0)),
            pl.BlockSpec((1, 1, M), lambda b: (b, 0, 0)),
            pl.BlockSpec((1, 1, N), lambda b: (b, 0, 0)),
        ],
        out_specs=pl.BlockSpec((1, M, N), lambda b: (b, 0, 0)),
        out_shape=jax.ShapeDtypeStruct((B, M, N), jnp.float32),
        compiler_params=pltpu.CompilerParams(
            dimension_semantics=("arbitrary",)),
    )
    confidence = p2(rowmax, rowarg, colmax)
    return scores, confidence
